# Initial kernel scaffold; baseline (speedup 1.0000x reference)
import jax
import jax.numpy as jnp
from jax.experimental import pallas as pl


def kernel(x, edge_index, bn1_gamma, bn1_beta, W1, b1, bn2_gamma, bn2_beta, W2, b2):
    raise NotImplementedError("write your pallas kernel here")


# speedup vs baseline: 11.3236x; 11.3236x over previous
import functools

import jax
import jax.numpy as jnp
from jax import lax
from jax.experimental import pallas as pl
from jax.experimental.pallas import tpu as pltpu
from jax.experimental.pallas import tpu_sc as plsc

N = 10000
E = 160000
D = 256
DH = D // 2
EPS = 1e-5

NC = 2
NS = 16
LANES = 16

NPAD = 10240
RPT = NPAD // NS

EPAD = 163840
CH = 128

NCH_A = EPAD // (NS * CH)
NCH_D = EPAD // (NC * NS * CH)

DEG_W = 128
ZR = 64

_MESH = plsc.VectorSubcoreMesh(core_axis_name="c", subcore_axis_name="s")


@functools.partial(
    pl.kernel,
    out_type=jax.ShapeDtypeStruct((NC, NPAD, DEG_W), jnp.float32),
    mesh=_MESH,
    scratch_types=[
        pltpu.VMEM((NCH_D, CH), jnp.int32),
        pltpu.VMEM((CH, DEG_W), jnp.float32),
        pltpu.VMEM_SHARED((NPAD, DEG_W), jnp.float32),
    ],
)
def _sc_deg(dst_hbm, z_hbm, ones_hbm, out_hbm, idxc, ones_v, acc_sh):
    c = lax.axis_index("c")
    s = lax.axis_index("s")
    row0 = (c * NS + s) * NCH_D

    def zb(k, carry):
        pltpu.sync_copy(z_hbm, acc_sh.at[pl.ds(s * RPT + k * ZR, ZR)])
        return carry

    lax.fori_loop(0, RPT // ZR, zb, 0)
    pltpu.sync_copy(ones_hbm, ones_v)
    pltpu.sync_copy(dst_hbm.at[pl.ds(row0, NCH_D)], idxc)

    plsc.subcore_barrier()

    def add_chunk(j, carry):
        pltpu.sync_copy(ones_v, acc_sh.at[idxc.at[j]], add=True)
        return carry

    lax.fori_loop(0, NCH_D, add_chunk, 0)

    plsc.subcore_barrier()

    pltpu.sync_copy(
        acc_sh.at[pl.ds(s * RPT, RPT)],
        out_hbm.at[c].at[pl.ds(s * RPT, RPT)],
    )


@functools.partial(
    pl.kernel,
    out_type=jax.ShapeDtypeStruct((NC, NPAD, DH), jnp.float32),
    mesh=_MESH,
    scratch_types=[
        pltpu.VMEM((NCH_A, CH), jnp.int32),
        pltpu.VMEM((NCH_A, CH), jnp.int32),
        pltpu.VMEM((CH, DH), jnp.float32),
        pltpu.VMEM_SHARED((NPAD, DH), jnp.float32),
        pltpu.SemaphoreType.DMA,
    ],
)
def _sc_agg(g_hbm, src_hbm, dst_hbm, out_hbm, srcc, dstc, rows_v, acc_sh, sem):
    c = lax.axis_index("c")
    s = lax.axis_index("s")
    row0 = s * NCH_A

    pltpu.sync_copy(
        g_hbm.at[c].at[pl.ds(s * RPT, RPT)],
        acc_sh.at[pl.ds(s * RPT, RPT)],
    )
    pltpu.sync_copy(src_hbm.at[pl.ds(row0, NCH_A)], srcc)
    pltpu.sync_copy(dst_hbm.at[pl.ds(row0, NCH_A)], dstc)
    plsc.subcore_barrier()

    def chunk(j, carry):
        pltpu.async_copy(g_hbm.at[c].at[srcc.at[j]], rows_v, sem).wait()
        pltpu.sync_copy(rows_v, acc_sh.at[dstc.at[j]], add=True)
        return carry

    lax.fori_loop(0, NCH_A, chunk, 0)

    plsc.subcore_barrier()

    pltpu.sync_copy(
        acc_sh.at[pl.ds(s * RPT, RPT)],
        out_hbm.at[c].at[pl.ds(s * RPT, RPT)],
    )


BR = 400
GRID_R = N // BR

_row_spec = pl.BlockSpec((BR, D), lambda i: (i, 0))
_half_spec = pl.BlockSpec((NC, BR, DH), lambda i: (0, i, 0))
_vec_spec = pl.BlockSpec((1, D), lambda i: (0, 0))
_w_spec = pl.BlockSpec((D, D), lambda i: (0, 0))
_deg_spec = pl.BlockSpec((NC, BR, DEG_W), lambda i: (0, i, 0))
_vec_shape = jax.ShapeDtypeStruct((1, D), jnp.float32)
_mat_shape = jax.ShapeDtypeStruct((N, D), jnp.float32)
_half_shape = jax.ShapeDtypeStruct((NC, NPAD, DH), jnp.float32)


def _dis_of(degp):
    deg = degp[0, :, 0:1] + degp[1, :, 0:1] + 1.0
    return lax.rsqrt(deg)


def _stats1_body(x_ref, s_ref, q_ref):
    i = pl.program_id(0)

    @pl.when(i == 0)
    def _():
        s_ref[...] = jnp.zeros_like(s_ref)
        q_ref[...] = jnp.zeros_like(q_ref)

    xb = x_ref[...]
    s_ref[...] += jnp.sum(xb, axis=0, keepdims=True)
    q_ref[...] += jnp.sum(xb * xb, axis=0, keepdims=True)


_stats1 = pl.pallas_call(
    _stats1_body,
    grid=(GRID_R,),
    in_specs=[_row_spec],
    out_specs=[_vec_spec, _vec_spec],
    out_shape=[_vec_shape, _vec_shape],
)


def _apply1_body(x_ref, s_ref, q_ref, g_ref, b_ref, w_ref, degp_ref, out_ref):
    mu = s_ref[...] * (1.0 / N)
    var = q_ref[...] * (1.0 / N) - mu * mu
    sc = g_ref[...] * lax.rsqrt(var + EPS)
    tn = (x_ref[...] - mu) * sc + b_ref[...]
    h = jnp.dot(tn, w_ref[...], preferred_element_type=jnp.float32)
    h = h * _dis_of(degp_ref[...])
    out_ref[0, :, :] = h[:, :DH]
    out_ref[1, :, :] = h[:, DH:]


_apply1 = pl.pallas_call(
    _apply1_body,
    grid=(GRID_R,),
    in_specs=[_row_spec, _vec_spec, _vec_spec, _vec_spec, _vec_spec, _w_spec,
              _deg_spec],
    out_specs=_half_spec,
    out_shape=_half_shape,
)


def _stats2_body(a_ref, degp_ref, s_ref, q_ref):
    i = pl.program_id(0)

    @pl.when(i == 0)
    def _():
        s_ref[...] = jnp.zeros_like(s_ref)
        q_ref[...] = jnp.zeros_like(q_ref)

    a = a_ref[...]
    v = jnp.concatenate([a[0], a[1]], axis=1) * _dis_of(degp_ref[...])
    s_ref[...] += jnp.sum(v, axis=0, keepdims=True)
    q_ref[...] += jnp.sum(v * v, axis=0, keepdims=True)


_stats2 = pl.pallas_call(
    _stats2_body,
    grid=(GRID_R,),
    in_specs=[_half_spec, _deg_spec],
    out_specs=[_vec_spec, _vec_spec],
    out_shape=[_vec_shape, _vec_shape],
)


def _apply2_body(a_ref, s_ref, q_ref, g_ref, b_ref, w_ref, degp_ref, out_ref):
    dis = _dis_of(degp_ref[...])
    a = a_ref[...]
    v = jnp.concatenate([a[0], a[1]], axis=1) * dis
    mu = s_ref[...] * (1.0 / N)
    var = q_ref[...] * (1.0 / N) - mu * mu
    sc = g_ref[...] * lax.rsqrt(var + EPS)
    tn = (v - mu) * sc + b_ref[...]
    h = jnp.dot(tn, w_ref[...], preferred_element_type=jnp.float32)
    h = h * dis
    out_ref[0, :, :] = h[:, :DH]
    out_ref[1, :, :] = h[:, DH:]


_apply2 = pl.pallas_call(
    _apply2_body,
    grid=(GRID_R,),
    in_specs=[_half_spec, _vec_spec, _vec_spec, _vec_spec, _vec_spec, _w_spec,
              _deg_spec],
    out_specs=_half_spec,
    out_shape=_half_shape,
)


def _final_body(a_ref, degp_ref, b_ref, out_ref):
    dis = _dis_of(degp_ref[...])
    a = a_ref[...]
    v = jnp.concatenate([a[0], a[1]], axis=1) * dis
    out_ref[...] = jnp.maximum(v + b_ref[...], 0.0)


_final = pl.pallas_call(
    _final_body,
    grid=(GRID_R,),
    in_specs=[_half_spec, _deg_spec, _vec_spec],
    out_specs=_row_spec,
    out_shape=_mat_shape,
)


def kernel(x, edge_index, bn1_gamma, bn1_beta, W1, b1, bn2_gamma, bn2_beta,
           W2, b2):
    del b1
    src = edge_index[0]
    dst = edge_index[1]
    padi = N + (jnp.arange(EPAD - E, dtype=jnp.int32) % (NPAD - N))
    srcp = jnp.concatenate([src, padi]).reshape(EPAD // CH, CH)
    dstp = jnp.concatenate([dst, padi]).reshape(EPAD // CH, CH)

    g1r = bn1_gamma.reshape(1, D)
    b1r = bn1_beta.reshape(1, D)
    g2r = bn2_gamma.reshape(1, D)
    b2r = bn2_beta.reshape(1, D)
    bias2 = b2.reshape(1, D)

    zrows = jnp.zeros((ZR, DEG_W), jnp.float32)
    orows = jnp.ones((CH, DEG_W), jnp.float32)
    degp = _sc_deg(dstp, zrows, orows)
    s1, q1 = _stats1(x)
    g1 = _apply1(x, s1, q1, g1r, b1r, W1, degp)
    a1 = _sc_agg(g1, srcp, dstp)
    s2, q2 = _stats2(a1, degp)
    g2 = _apply2(a1, s2, q2, g2r, b2r, W2, degp)
    a2 = _sc_agg(g2, srcp, dstp)
    return _final(a2, degp, bias2)


# speedup vs baseline: 11.8240x; 1.0442x over previous
import functools

import jax
import jax.numpy as jnp
from jax import lax
from jax.experimental import pallas as pl
from jax.experimental.pallas import tpu as pltpu
from jax.experimental.pallas import tpu_sc as plsc

N = 10000
E = 160000
D = 256
DH = D // 2
EPS = 1e-5

NC = 2
NS = 16
LANES = 16

NPAD = 10240
RPT = NPAD // NS

EPAD = 163840
CH = 128

NCH_A = EPAD // (NS * CH)
NCH_D = EPAD // (NC * NS * CH)

DEG_W = 16
ZR = 64

_MESH = plsc.VectorSubcoreMesh(core_axis_name="c", subcore_axis_name="s")


@functools.partial(
    pl.kernel,
    out_type=jax.ShapeDtypeStruct((NC, NPAD, DEG_W), jnp.float32),
    mesh=_MESH,
    scratch_types=[
        pltpu.VMEM((NCH_D, CH), jnp.int32),
        pltpu.VMEM((CH, DEG_W), jnp.float32),
        pltpu.VMEM_SHARED((NPAD, DEG_W), jnp.float32),
    ],
)
def _sc_deg(dst_hbm, z_hbm, ones_hbm, out_hbm, idxc, ones_v, acc_sh):
    c = lax.axis_index("c")
    s = lax.axis_index("s")
    row0 = (c * NS + s) * NCH_D

    def zb(k, carry):
        pltpu.sync_copy(z_hbm, acc_sh.at[pl.ds(s * RPT + k * ZR, ZR)])
        return carry

    lax.fori_loop(0, RPT // ZR, zb, 0)
    pltpu.sync_copy(ones_hbm, ones_v)
    pltpu.sync_copy(dst_hbm.at[pl.ds(row0, NCH_D)], idxc)

    plsc.subcore_barrier()

    def add_chunk(j, carry):
        pltpu.sync_copy(ones_v, acc_sh.at[idxc.at[j]], add=True)
        return carry

    lax.fori_loop(0, NCH_D, add_chunk, 0)

    plsc.subcore_barrier()

    pltpu.sync_copy(
        acc_sh.at[pl.ds(s * RPT, RPT)],
        out_hbm.at[c].at[pl.ds(s * RPT, RPT)],
    )


@functools.partial(
    pl.kernel,
    out_type=jax.ShapeDtypeStruct((NC, NPAD, DH), jnp.float32),
    mesh=_MESH,
    scratch_types=[
        pltpu.VMEM((NCH_A, CH), jnp.int32),
        pltpu.VMEM((NCH_A, CH), jnp.int32),
        pltpu.VMEM((CH, DH), jnp.float32),
        pltpu.VMEM_SHARED((NPAD, DH), jnp.float32),
        pltpu.SemaphoreType.DMA,
    ],
)
def _sc_agg(g_hbm, src_hbm, dst_hbm, out_hbm, srcc, dstc, rows, acc_sh, sem):
    c = lax.axis_index("c")
    s = lax.axis_index("s")
    row0 = s * NCH_A

    pltpu.sync_copy(
        g_hbm.at[c].at[pl.ds(s * RPT, RPT)],
        acc_sh.at[pl.ds(s * RPT, RPT)],
    )
    pltpu.sync_copy(src_hbm.at[pl.ds(row0, NCH_A)], srcc)
    pltpu.sync_copy(dst_hbm.at[pl.ds(row0, NCH_A)], dstc)
    plsc.subcore_barrier()

    def chunk(j, carry):
        pltpu.async_copy(g_hbm.at[c].at[srcc.at[j]], rows, sem).wait()
        pltpu.sync_copy(rows, acc_sh.at[dstc.at[j]], add=True)
        return carry

    lax.fori_loop(0, NCH_A, chunk, 0)

    plsc.subcore_barrier()

    pltpu.sync_copy(
        acc_sh.at[pl.ds(s * RPT, RPT)],
        out_hbm.at[c].at[pl.ds(s * RPT, RPT)],
    )


BR = 400
GRID_R = N // BR

_row_spec = pl.BlockSpec((BR, D), lambda i: (i, 0))
_half_spec = pl.BlockSpec((NC, BR, DH), lambda i: (0, i, 0))
_vec_spec = pl.BlockSpec((1, D), lambda i: (0, 0))
_w_spec = pl.BlockSpec((D, D), lambda i: (0, 0))
_deg_spec = pl.BlockSpec((NC, BR, DEG_W), lambda i: (0, i, 0))
_vec_shape = jax.ShapeDtypeStruct((1, D), jnp.float32)
_mat_shape = jax.ShapeDtypeStruct((N, D), jnp.float32)
_half_shape = jax.ShapeDtypeStruct((NC, NPAD, DH), jnp.float32)


def _dis_of(degp):
    deg = degp[0, :, 0:1] + degp[1, :, 0:1] + 1.0
    return lax.rsqrt(deg)


def _stats1_body(x_ref, s_ref, q_ref):
    i = pl.program_id(0)

    @pl.when(i == 0)
    def _():
        s_ref[...] = jnp.zeros_like(s_ref)
        q_ref[...] = jnp.zeros_like(q_ref)

    xb = x_ref[...]
    s_ref[...] += jnp.sum(xb, axis=0, keepdims=True)
    q_ref[...] += jnp.sum(xb * xb, axis=0, keepdims=True)


_stats1 = pl.pallas_call(
    _stats1_body,
    grid=(GRID_R,),
    in_specs=[_row_spec],
    out_specs=[_vec_spec, _vec_spec],
    out_shape=[_vec_shape, _vec_shape],
)


def _apply1_body(x_ref, s_ref, q_ref, g_ref, b_ref, w_ref, degp_ref, out_ref):
    mu = s_ref[...] * (1.0 / N)
    var = q_ref[...] * (1.0 / N) - mu * mu
    sc = g_ref[...] * lax.rsqrt(var + EPS)
    tn = (x_ref[...] - mu) * sc + b_ref[...]
    h = jnp.dot(tn, w_ref[...], preferred_element_type=jnp.float32)
    h = h * _dis_of(degp_ref[...])
    out_ref[0, :, :] = h[:, :DH]
    out_ref[1, :, :] = h[:, DH:]


_apply1 = pl.pallas_call(
    _apply1_body,
    grid=(GRID_R,),
    in_specs=[_row_spec, _vec_spec, _vec_spec, _vec_spec, _vec_spec, _w_spec,
              _deg_spec],
    out_specs=_half_spec,
    out_shape=_half_shape,
)


def _stats2_body(a_ref, degp_ref, s_ref, q_ref):
    i = pl.program_id(0)

    @pl.when(i == 0)
    def _():
        s_ref[...] = jnp.zeros_like(s_ref)
        q_ref[...] = jnp.zeros_like(q_ref)

    a = a_ref[...]
    v = jnp.concatenate([a[0], a[1]], axis=1) * _dis_of(degp_ref[...])
    s_ref[...] += jnp.sum(v, axis=0, keepdims=True)
    q_ref[...] += jnp.sum(v * v, axis=0, keepdims=True)


_stats2 = pl.pallas_call(
    _stats2_body,
    grid=(GRID_R,),
    in_specs=[_half_spec, _deg_spec],
    out_specs=[_vec_spec, _vec_spec],
    out_shape=[_vec_shape, _vec_shape],
)


def _apply2_body(a_ref, s_ref, q_ref, g_ref, b_ref, w_ref, degp_ref, out_ref):
    dis = _dis_of(degp_ref[...])
    a = a_ref[...]
    v = jnp.concatenate([a[0], a[1]], axis=1) * dis
    mu = s_ref[...] * (1.0 / N)
    var = q_ref[...] * (1.0 / N) - mu * mu
    sc = g_ref[...] * lax.rsqrt(var + EPS)
    tn = (v - mu) * sc + b_ref[...]
    h = jnp.dot(tn, w_ref[...], preferred_element_type=jnp.float32)
    h = h * dis
    out_ref[0, :, :] = h[:, :DH]
    out_ref[1, :, :] = h[:, DH:]


_apply2 = pl.pallas_call(
    _apply2_body,
    grid=(GRID_R,),
    in_specs=[_half_spec, _vec_spec, _vec_spec, _vec_spec, _vec_spec, _w_spec,
              _deg_spec],
    out_specs=_half_spec,
    out_shape=_half_shape,
)


def _final_body(a_ref, degp_ref, b_ref, out_ref):
    dis = _dis_of(degp_ref[...])
    a = a_ref[...]
    v = jnp.concatenate([a[0], a[1]], axis=1) * dis
    out_ref[...] = jnp.maximum(v + b_ref[...], 0.0)


_final = pl.pallas_call(
    _final_body,
    grid=(GRID_R,),
    in_specs=[_half_spec, _deg_spec, _vec_spec],
    out_specs=_row_spec,
    out_shape=_mat_shape,
)


def kernel(x, edge_index, bn1_gamma, bn1_beta, W1, b1, bn2_gamma, bn2_beta,
           W2, b2):
    del b1
    src = edge_index[0]
    dst = edge_index[1]
    padi = N + (jnp.arange(EPAD - E, dtype=jnp.int32) % (NPAD - N))
    srcp = jnp.concatenate([src, padi]).reshape(EPAD // CH, CH)
    dstp = jnp.concatenate([dst, padi]).reshape(EPAD // CH, CH)

    g1r = bn1_gamma.reshape(1, D)
    b1r = bn1_beta.reshape(1, D)
    g2r = bn2_gamma.reshape(1, D)
    b2r = bn2_beta.reshape(1, D)
    bias2 = b2.reshape(1, D)

    zrows = jnp.zeros((ZR, DEG_W), jnp.float32)
    orows = jnp.ones((CH, DEG_W), jnp.float32)
    degp = _sc_deg(dstp, zrows, orows)
    s1, q1 = _stats1(x)
    g1 = _apply1(x, s1, q1, g1r, b1r, W1, degp)
    a1 = _sc_agg(g1, srcp, dstp)
    s2, q2 = _stats2(a1, degp)
    g2 = _apply2(a1, s2, q2, g2r, b2r, W2, degp)
    a2 = _sc_agg(g2, srcp, dstp)
    return _final(a2, degp, bias2)
